# baseline (speedup 1.0000x reference)
import jax
import jax.numpy as jnp
from jax import lax
from jax.experimental import pallas as pl
from jax.experimental.pallas import tpu as pltpu
from jax.experimental.pallas import tpu_sc as plsc

HIDDEN = 512
NUM_CORES = 2
NUM_SUBCORES = 16
NW = NUM_CORES * NUM_SUBCORES
GROUP = 8
ROW_W = GROUP * HIDDEN
GPD = 8


def _sc_body(table_hbm, idx_hbm, out_hbm,
             raw_v, packed_v, rows0, rows1, gsem0, gsem1, ssem0, ssem1):
    wid = lax.axis_index("s") * NUM_CORES + lax.axis_index("c")
    n_idx = idx_hbm.shape[0]
    idx_per_w = n_idx // NW
    grp_per_w = idx_per_w // GROUP
    base_g = wid * grp_per_w

    rows = (rows0, rows1)
    gsem = (gsem0, gsem1)
    ssem = (ssem0, ssem1)

    pltpu.sync_copy(idx_hbm.at[pl.ds(wid * idx_per_w, idx_per_w)], raw_v)

    def pack_step(j, carry):
        sl = pl.ds(j * 16, 16)
        p = raw_v[sl]
        for k in range(1, GROUP):
            p = p + p + raw_v[pl.ds(k * grp_per_w + j * 16, 16)]
        packed_v[sl] = p
        return carry

    lax.fori_loop(0, grp_per_w // 16, pack_step, 0)

    n_chunks = grp_per_w // GPD

    def start_gather(b, c):
        pltpu.async_copy(
            table_hbm.at[packed_v.at[pl.ds(c * GPD, GPD)]], rows[b], gsem[b])

    def wait_gather(b, c):
        pltpu.make_async_copy(
            table_hbm.at[packed_v.at[pl.ds(c * GPD, GPD)]], rows[b], gsem[b]
        ).wait()

    def start_scatter(b, c):
        pltpu.async_copy(
            rows[b], out_hbm.at[pl.ds(base_g + c * GPD, GPD)], ssem[b])

    def wait_scatter(b, c):
        pltpu.make_async_copy(
            rows[b], out_hbm.at[pl.ds(base_g + c * GPD, GPD)], ssem[b]).wait()

    start_gather(0, 0)
    start_gather(1, 1)

    def step(j, carry):
        for b in range(2):
            c = 2 * j + b
            wait_gather(b, c)
            start_scatter(b, c)

            @pl.when(j < (n_chunks // 2) - 1)
            def _():
                wait_scatter(b, c)
                start_gather(b, c + 2)
        return carry

    lax.fori_loop(0, n_chunks // 2, step, 0)
    wait_scatter(0, n_chunks - 2)
    wait_scatter(1, n_chunks - 1)


def kernel(inputs, shared_weights):
    B, S = inputs.shape
    n_rows = B * S
    n_groups = n_rows // GROUP
    table2 = shared_weights.at[0].set(0.0) * (HIDDEN ** 0.5)
    bits = (jnp.arange(256, dtype=jnp.int32)[:, None]
            >> (GROUP - 1 - jnp.arange(GROUP, dtype=jnp.int32))[None, :]) & 1
    table8 = table2[bits].reshape(256, ROW_W)
    idx = (inputs.reshape(NW, n_groups // NW, GROUP)
           .transpose(0, 2, 1).reshape(n_rows).astype(jnp.int32))

    mesh = plsc.VectorSubcoreMesh(core_axis_name="c", subcore_axis_name="s")
    sc_call = pl.kernel(
        _sc_body,
        out_type=jax.ShapeDtypeStruct((n_groups, ROW_W), jnp.float32),
        mesh=mesh,
        scratch_types=[
            pltpu.VMEM((n_rows // NW,), jnp.int32),
            pltpu.VMEM((n_groups // NW,), jnp.int32),
            pltpu.VMEM((GPD, ROW_W), jnp.float32),
            pltpu.VMEM((GPD, ROW_W), jnp.float32),
            pltpu.SemaphoreType.DMA,
            pltpu.SemaphoreType.DMA,
            pltpu.SemaphoreType.DMA,
            pltpu.SemaphoreType.DMA,
        ],
    )
    out = sc_call(table8, idx)
    return out.reshape(B, S, HIDDEN)


# speedup vs baseline: 5.6704x; 5.6704x over previous
import jax
import jax.numpy as jnp
from jax import lax
from jax.experimental import pallas as pl
from jax.experimental.pallas import tpu as pltpu
from jax.experimental.pallas import tpu_sc as plsc

HIDDEN = 512
NUM_CORES = 2
NUM_SUBCORES = 16
NW = NUM_CORES * NUM_SUBCORES
CHUNK = 64
NLANE = 16
MREG = HIDDEN // NLANE


def _sc_body(w_hbm, idx_hbm, out_hbm,
             raw_v, w_v, rows0, rows1, ssem0, ssem1):
    wid = lax.axis_index("s") * NUM_CORES + lax.axis_index("c")
    n_rows = idx_hbm.shape[0]
    rows_per_w = n_rows // NW
    base = wid * rows_per_w
    n_chunks = rows_per_w // CHUNK

    rows = (rows0, rows1)
    ssem = (ssem0, ssem1)

    pltpu.sync_copy(idx_hbm.at[pl.ds(base, rows_per_w)], raw_v)
    pltpu.sync_copy(w_hbm, w_v)
    wregs = tuple(w_v[pl.ds(m * NLANE, NLANE)] for m in range(MREG))

    def fill(b, c):
        def tstep(t, carry):
            xv = raw_v[pl.ds(c * CHUNK + t * NLANE, NLANE)].astype(jnp.float32)
            for l in range(NLANE):
                xb = jnp.broadcast_to(xv[l], (NLANE,))
                r = t * NLANE + l
                for m in range(MREG):
                    rows[b][r, pl.ds(m * NLANE, NLANE)] = wregs[m] * xb
            return carry

        lax.fori_loop(0, CHUNK // NLANE, tstep, 0)

    def start_scatter(b, c):
        pltpu.async_copy(
            rows[b], out_hbm.at[pl.ds(base + c * CHUNK, CHUNK)], ssem[b])

    def wait_scatter(b, c):
        pltpu.make_async_copy(
            rows[b], out_hbm.at[pl.ds(base + c * CHUNK, CHUNK)], ssem[b]).wait()

    fill(0, 0)
    start_scatter(0, 0)
    fill(1, 1)
    start_scatter(1, 1)

    def step(j, carry):
        for b in range(2):
            c = 2 * j + b

            @pl.when(j < (n_chunks // 2) - 1)
            def _():
                wait_scatter(b, c)
                fill(b, c + 2)
                start_scatter(b, c + 2)
        return carry

    lax.fori_loop(0, n_chunks // 2, step, 0)
    wait_scatter(0, n_chunks - 2)
    wait_scatter(1, n_chunks - 1)


def kernel(inputs, shared_weights):
    B, S = inputs.shape
    n_rows = B * S
    w_scaled = shared_weights[1] * (HIDDEN ** 0.5)
    idx = inputs.reshape(n_rows).astype(jnp.int32)

    mesh = plsc.VectorSubcoreMesh(core_axis_name="c", subcore_axis_name="s")
    sc_call = pl.kernel(
        _sc_body,
        out_type=jax.ShapeDtypeStruct((n_rows, HIDDEN), jnp.float32),
        mesh=mesh,
        scratch_types=[
            pltpu.VMEM((n_rows // NW,), jnp.int32),
            pltpu.VMEM((HIDDEN,), jnp.float32),
            pltpu.VMEM((CHUNK, HIDDEN), jnp.float32),
            pltpu.VMEM((CHUNK, HIDDEN), jnp.float32),
            pltpu.SemaphoreType.DMA,
            pltpu.SemaphoreType.DMA,
        ],
    )
    out = sc_call(w_scaled, idx)
    return out.reshape(B, S, HIDDEN)
